# baseline (speedup 1.0000x reference)
import functools

import jax
import jax.numpy as jnp
from jax import lax
from jax.experimental import pallas as pl
from jax.experimental.pallas import tpu as pltpu
from jax.experimental.pallas import tpu_sc as plsc

NC = 2
NS = 16
CHUNK = 128
DEG_W = 16
ROW_BLK = 1024


def _sc_agg(n_pad, chunks_per_worker):
    rpt = n_pad // NS
    cpw = chunks_per_worker

    slab = cpw // 2

    def body(y_hbm, src_hbm, dst_hbm, zeros_hbm, out0, out1,
             src_slab, dst0, dst1, buf0, buf1, acc, gsem, dsem):
        c = lax.axis_index("c")
        s = lax.axis_index("s")
        w = c * NS + s
        pltpu.sync_copy(zeros_hbm, acc.at[pl.ds(s * rpt, rpt)])
        plsc.subcore_barrier()

        bufs = (buf0, buf1)
        dsts = (dst0, dst1)

        def half(h, carry):
            base = w * cpw + h * slab
            pltpu.sync_copy(src_hbm.at[pl.ds(base, slab)], src_slab)
            pltpu.async_copy(dst_hbm.at[base], dst0, dsem)
            pltpu.async_copy(y_hbm.at[src_slab.at[0]], buf0, gsem)

            def step(g, carry2):
                for b in range(2):
                    j = 2 * g + b
                    pltpu.make_async_copy(y_hbm.at[src_slab.at[j]],
                                          bufs[b], gsem).wait()
                    pltpu.make_async_copy(dst_hbm.at[base], dsts[b],
                                          dsem).wait()

                    @pl.when(j + 1 < slab)
                    def _():
                        pltpu.async_copy(y_hbm.at[src_slab.at[j + 1]],
                                         bufs[1 - b], gsem)
                        pltpu.async_copy(dst_hbm.at[base + j + 1],
                                         dsts[1 - b], dsem)

                    pltpu.sync_copy(bufs[b], acc.at[dsts[b]], add=True)
                return carry2

            lax.fori_loop(0, slab // 2, step, 0)
            return carry

        lax.fori_loop(0, cpw // slab, half, 0)
        plsc.subcore_barrier()

        @pl.when(c == 0)
        def _():
            pltpu.sync_copy(acc.at[pl.ds(s * rpt, rpt)],
                            out0.at[pl.ds(s * rpt, rpt)])

        @pl.when(c == 1)
        def _():
            pltpu.sync_copy(acc.at[pl.ds(s * rpt, rpt)],
                            out1.at[pl.ds(s * rpt, rpt)])

    shape = jax.ShapeDtypeStruct((n_pad, 128), jnp.float32)
    return pl.kernel(
        body,
        mesh=plsc.VectorSubcoreMesh(core_axis_name="c", subcore_axis_name="s"),
        out_type=(shape, shape),
        scratch_types=[
            pltpu.VMEM((slab, CHUNK), jnp.int32),
            pltpu.VMEM((CHUNK,), jnp.int32),
            pltpu.VMEM((CHUNK,), jnp.int32),
            pltpu.VMEM((CHUNK, 128), jnp.float32),
            pltpu.VMEM((CHUNK, 128), jnp.float32),
            pltpu.VMEM_SHARED((n_pad, 128), jnp.float32),
            pltpu.SemaphoreType.DMA,
            pltpu.SemaphoreType.DMA,
        ],
    )


def _sc_deg(n_pad, chunks_per_worker):
    rpt = n_pad // NS

    cpw = chunks_per_worker

    def body(dst_hbm, ones_hbm, zeros_hbm, out0, out1,
             ones_v, dst0, dst1, acc, dsem):
        c = lax.axis_index("c")
        s = lax.axis_index("s")
        w = c * NS + s
        pltpu.sync_copy(zeros_hbm, acc.at[pl.ds(s * rpt, rpt)])
        pltpu.sync_copy(ones_hbm, ones_v)
        base = w * cpw
        dsts = (dst0, dst1)
        pltpu.async_copy(dst_hbm.at[base], dst0, dsem)
        plsc.subcore_barrier()

        def step(g, carry):
            for b in range(2):
                j = 2 * g + b
                pltpu.make_async_copy(dst_hbm.at[base], dsts[b], dsem).wait()

                @pl.when(j + 1 < cpw)
                def _():
                    pltpu.async_copy(dst_hbm.at[base + j + 1], dsts[1 - b],
                                     dsem)

                pltpu.sync_copy(ones_v, acc.at[dsts[b]], add=True)
            return carry

        lax.fori_loop(0, cpw // 2, step, 0)
        plsc.subcore_barrier()

        @pl.when(c == 0)
        def _():
            pltpu.sync_copy(acc.at[pl.ds(s * rpt, rpt)],
                            out0.at[pl.ds(s * rpt, rpt)])

        @pl.when(c == 1)
        def _():
            pltpu.sync_copy(acc.at[pl.ds(s * rpt, rpt)],
                            out1.at[pl.ds(s * rpt, rpt)])

    shape = jax.ShapeDtypeStruct((n_pad, DEG_W), jnp.float32)
    return pl.kernel(
        body,
        mesh=plsc.VectorSubcoreMesh(core_axis_name="c", subcore_axis_name="s"),
        out_type=(shape, shape),
        scratch_types=[
            pltpu.VMEM((CHUNK, DEG_W), jnp.float32),
            pltpu.VMEM((CHUNK,), jnp.int32),
            pltpu.VMEM((CHUNK,), jnp.int32),
            pltpu.VMEM_SHARED((n_pad, DEG_W), jnp.float32),
            pltpu.SemaphoreType.DMA,
        ],
    )


def _dinv_block(d0_ref, d1_ref):
    deg = 1.0 + d0_ref[...][:, :1] + d1_ref[...][:, :1]
    return lax.rsqrt(deg)


def _mm_body(x_ref, w_ref, o_ref):
    o_ref[...] = jnp.dot(x_ref[...], w_ref[...],
                         preferred_element_type=jnp.float32)


def _prep_body(d0_ref, d1_ref, xw_ref, y_ref):
    y_ref[...] = _dinv_block(d0_ref, d1_ref) * xw_ref[...]


def _fuse1_body(d0_ref, d1_ref, p0_ref, p1_ref, xw1_ref, b1_ref, w2_ref,
                xw2_ref, y2_ref):
    dinv = _dinv_block(d0_ref, d1_ref)
    h = jnp.maximum(
        dinv * (p0_ref[...] + p1_ref[...])
        + (dinv * dinv) * xw1_ref[...] + b1_ref[...], 0.0)
    xw2 = jnp.dot(h, w2_ref[...], preferred_element_type=jnp.float32)
    xw2_ref[...] = xw2
    y2_ref[...] = dinv * xw2


def _fuse2_body(d0_ref, d1_ref, p0_ref, p1_ref, xw2_ref, b2_ref, o_ref):
    dinv = _dinv_block(d0_ref, d1_ref)
    o_ref[...] = (dinv * (p0_ref[...] + p1_ref[...])
                  + (dinv * dinv) * xw2_ref[...] + b2_ref[...])


def _row_spec(width):
    return pl.BlockSpec((ROW_BLK, width), lambda i: (i, 0))


def _full_spec(shape):
    return pl.BlockSpec(shape, lambda i: tuple(0 for _ in shape))


def kernel(x, edge_index, W1, b1, W2, b2):
    n, d_in = x.shape
    e = edge_index.shape[1]
    n_pad = ((n + 1 + ROW_BLK - 1) // ROW_BLK) * ROW_BLK
    cpw = -(-e // (NC * NS * CHUNK))
    cpw = ((cpw + 3) // 4) * 4
    e_pad = NC * NS * CHUNK * cpw
    grid = (n_pad // ROW_BLK,)

    src = jnp.concatenate(
        [edge_index[0], jnp.full((e_pad - e,), n, jnp.int32)]
    ).reshape(-1, CHUNK)
    dst = jnp.concatenate(
        [edge_index[1], jnp.full((e_pad - e,), n, jnp.int32)]
    ).reshape(-1, CHUNK)
    x_p = jnp.pad(x, ((0, n_pad - n), (0, 0)))
    b1r = b1.reshape(1, -1)
    b2r = b2.reshape(1, -1)

    zeros_deg = jnp.zeros((n_pad // NS, DEG_W), jnp.float32)
    ones_deg = jnp.ones((CHUNK, DEG_W), jnp.float32)
    zeros_agg = jnp.zeros((n_pad // NS, 128), jnp.float32)

    d0, d1 = _sc_deg(n_pad, cpw)(dst, ones_deg, zeros_deg)

    xw1 = pl.pallas_call(
        _mm_body, grid=grid,
        in_specs=[_row_spec(d_in), _full_spec(W1.shape)],
        out_specs=_row_spec(W1.shape[1]),
        out_shape=jax.ShapeDtypeStruct((n_pad, W1.shape[1]), jnp.float32),
    )(x_p, W1)

    dspec = _row_spec(DEG_W)
    y1 = pl.pallas_call(
        _prep_body, grid=grid,
        in_specs=[dspec, dspec, _row_spec(128)],
        out_specs=_row_spec(128),
        out_shape=jax.ShapeDtypeStruct((n_pad, 128), jnp.float32),
    )(d0, d1, xw1)

    agg = _sc_agg(n_pad, cpw)
    p0, p1 = agg(y1, src, dst, zeros_agg)

    xw2, y2 = pl.pallas_call(
        _fuse1_body, grid=grid,
        in_specs=[dspec, dspec, _row_spec(128), _row_spec(128), _row_spec(128),
                  _full_spec(b1r.shape), _full_spec(W2.shape)],
        out_specs=(_row_spec(128), _row_spec(128)),
        out_shape=(jax.ShapeDtypeStruct((n_pad, 128), jnp.float32),
                   jax.ShapeDtypeStruct((n_pad, 128), jnp.float32)),
    )(d0, d1, p0, p1, xw1, b1r, W2)

    q0, q1 = agg(y2, src, dst, zeros_agg)

    out = pl.pallas_call(
        _fuse2_body, grid=grid,
        in_specs=[dspec, dspec, _row_spec(128), _row_spec(128), _row_spec(128),
                  _full_spec(b2r.shape)],
        out_specs=_row_spec(128),
        out_shape=jax.ShapeDtypeStruct((n_pad, 128), jnp.float32),
    )(d0, d1, q0, q1, xw2, b2r)

    return out[:n]


# speedup vs baseline: 2.9306x; 2.9306x over previous
import functools

import jax
import jax.numpy as jnp
from jax import lax
from jax.experimental import pallas as pl
from jax.experimental.pallas import tpu as pltpu
from jax.experimental.pallas import tpu_sc as plsc

NC = 2
NS = 16
CHUNK = 128
DEG_W = 16
ROW_BLK = 1024


def _sc_agg(n_pad, chunks_per_worker):
    rpt = n_pad // NS
    cpw = chunks_per_worker

    slab = cpw // 2

    def body(y_hbm, src_hbm, dst_hbm, zeros_hbm, out0, out1,
             src_slab, dst0, dst1, buf0, buf1, acc, gsem, dsem):
        c = lax.axis_index("c")
        s = lax.axis_index("s")
        w = c * NS + s
        pltpu.sync_copy(zeros_hbm, acc.at[pl.ds(s * rpt, rpt)])
        plsc.subcore_barrier()

        bufs = (buf0, buf1)
        dsts = (dst0, dst1)

        def half(h, carry):
            base = w * cpw + h * slab
            pltpu.sync_copy(src_hbm.at[pl.ds(base, slab)], src_slab)
            pltpu.async_copy(dst_hbm.at[base], dst0, dsem)
            pltpu.async_copy(y_hbm.at[src_slab.at[0]], buf0, gsem)

            def step(g, carry2):
                for b in range(2):
                    j = 2 * g + b
                    pltpu.make_async_copy(y_hbm.at[src_slab.at[j]],
                                          bufs[b], gsem).wait()
                    pltpu.make_async_copy(dst_hbm.at[base], dsts[b],
                                          dsem).wait()

                    @pl.when(j + 1 < slab)
                    def _():
                        pltpu.async_copy(y_hbm.at[src_slab.at[j + 1]],
                                         bufs[1 - b], gsem)
                        pltpu.async_copy(dst_hbm.at[base + j + 1],
                                         dsts[1 - b], dsem)

                    pltpu.sync_copy(bufs[b], acc.at[dsts[b]], add=True)
                return carry2

            lax.fori_loop(0, slab // 2, step, 0)
            return carry

        lax.fori_loop(0, cpw // slab, half, 0)
        plsc.subcore_barrier()

        @pl.when(c == 0)
        def _():
            pltpu.sync_copy(acc.at[pl.ds(s * rpt, rpt)],
                            out0.at[pl.ds(s * rpt, rpt)])

        @pl.when(c == 1)
        def _():
            pltpu.sync_copy(acc.at[pl.ds(s * rpt, rpt)],
                            out1.at[pl.ds(s * rpt, rpt)])

    shape = jax.ShapeDtypeStruct((n_pad, 128), jnp.float32)
    return pl.kernel(
        body,
        mesh=plsc.VectorSubcoreMesh(core_axis_name="c", subcore_axis_name="s"),
        out_type=(shape, shape),
        scratch_types=[
            pltpu.VMEM((slab, CHUNK), jnp.int32),
            pltpu.VMEM((CHUNK,), jnp.int32),
            pltpu.VMEM((CHUNK,), jnp.int32),
            pltpu.VMEM((CHUNK, 128), jnp.float32),
            pltpu.VMEM((CHUNK, 128), jnp.float32),
            pltpu.VMEM_SHARED((n_pad, 128), jnp.float32),
            pltpu.SemaphoreType.DMA,
            pltpu.SemaphoreType.DMA,
        ],
    )


def _sc_deg(n_pad, chunks_per_worker):
    rpt = n_pad // NS

    cpw = chunks_per_worker

    def body(dst_hbm, ones_hbm, zeros_hbm, out0, out1,
             ones_v, dst0, dst1, acc, dsem):
        c = lax.axis_index("c")
        s = lax.axis_index("s")
        w = c * NS + s
        pltpu.sync_copy(zeros_hbm, acc.at[pl.ds(s * rpt, rpt)])
        pltpu.sync_copy(ones_hbm, ones_v)
        base = w * cpw
        dsts = (dst0, dst1)
        pltpu.async_copy(dst_hbm.at[base], dst0, dsem)
        plsc.subcore_barrier()

        def step(g, carry):
            for b in range(2):
                j = 2 * g + b
                pltpu.make_async_copy(dst_hbm.at[base], dsts[b], dsem).wait()

                @pl.when(j + 1 < cpw)
                def _():
                    pltpu.async_copy(dst_hbm.at[base + j + 1], dsts[1 - b],
                                     dsem)

                pltpu.sync_copy(ones_v, acc.at[dsts[b]], add=True)
            return carry

        lax.fori_loop(0, cpw // 2, step, 0)
        plsc.subcore_barrier()

        @pl.when(c == 0)
        def _():
            pltpu.sync_copy(acc.at[pl.ds(s * rpt, rpt)],
                            out0.at[pl.ds(s * rpt, rpt)])

        @pl.when(c == 1)
        def _():
            pltpu.sync_copy(acc.at[pl.ds(s * rpt, rpt)],
                            out1.at[pl.ds(s * rpt, rpt)])

    shape = jax.ShapeDtypeStruct((n_pad, DEG_W), jnp.float32)
    return pl.kernel(
        body,
        mesh=plsc.VectorSubcoreMesh(core_axis_name="c", subcore_axis_name="s"),
        out_type=(shape, shape),
        scratch_types=[
            pltpu.VMEM((CHUNK, DEG_W), jnp.float32),
            pltpu.VMEM((CHUNK,), jnp.int32),
            pltpu.VMEM((CHUNK,), jnp.int32),
            pltpu.VMEM_SHARED((n_pad, DEG_W), jnp.float32),
            pltpu.SemaphoreType.DMA,
        ],
    )


def _dinv_block(d0_ref, d1_ref):
    deg = 1.0 + d0_ref[...][:, :1] + d1_ref[...][:, :1]
    return lax.rsqrt(deg)


def _mm_body(x_ref, w_ref, o_ref):
    o_ref[...] = jnp.dot(x_ref[...], w_ref[...],
                         preferred_element_type=jnp.float32)


def _prep_body(d0_ref, d1_ref, xw_ref, y_ref):
    y_ref[...] = _dinv_block(d0_ref, d1_ref) * xw_ref[...]


def _fuse1_body(d0_ref, d1_ref, p0_ref, p1_ref, xw1_ref, b1_ref, w2_ref,
                xw2_ref, y2_ref):
    dinv = _dinv_block(d0_ref, d1_ref)
    h = jnp.maximum(
        dinv * (p0_ref[...] + p1_ref[...])
        + (dinv * dinv) * xw1_ref[...] + b1_ref[...], 0.0)
    xw2 = jnp.dot(h, w2_ref[...], preferred_element_type=jnp.float32)
    xw2_ref[...] = xw2
    y2_ref[...] = dinv * xw2


def _fuse2_body(d0_ref, d1_ref, p0_ref, p1_ref, xw2_ref, b2_ref, o_ref):
    dinv = _dinv_block(d0_ref, d1_ref)
    o_ref[...] = (dinv * (p0_ref[...] + p1_ref[...])
                  + (dinv * dinv) * xw2_ref[...] + b2_ref[...])


def _row_spec(width):
    return pl.BlockSpec((ROW_BLK, width), lambda i: (i, 0))


def _full_spec(shape):
    return pl.BlockSpec(shape, lambda i: tuple(0 for _ in shape))


def kernel(x, edge_index, W1, b1, W2, b2):
    n, d_in = x.shape
    e = edge_index.shape[1]
    n_pad = ((n + 1 + ROW_BLK - 1) // ROW_BLK) * ROW_BLK
    cpw = -(-e // (NC * NS * CHUNK))
    cpw = ((cpw + 3) // 4) * 4
    e_pad = NC * NS * CHUNK * cpw
    grid = (n_pad // ROW_BLK,)

    pad_idx = n + jnp.arange(e_pad - e, dtype=jnp.int32) % (n_pad - n)
    src = jnp.concatenate([edge_index[0], pad_idx]).reshape(-1, CHUNK)
    dst = jnp.concatenate([edge_index[1], pad_idx]).reshape(-1, CHUNK)
    x_p = jnp.pad(x, ((0, n_pad - n), (0, 0)))
    b1r = b1.reshape(1, -1)
    b2r = b2.reshape(1, -1)

    zeros_deg = jnp.zeros((n_pad // NS, DEG_W), jnp.float32)
    ones_deg = jnp.ones((CHUNK, DEG_W), jnp.float32)
    zeros_agg = jnp.zeros((n_pad // NS, 128), jnp.float32)

    d0, d1 = _sc_deg(n_pad, cpw)(dst, ones_deg, zeros_deg)

    xw1 = pl.pallas_call(
        _mm_body, grid=grid,
        in_specs=[_row_spec(d_in), _full_spec(W1.shape)],
        out_specs=_row_spec(W1.shape[1]),
        out_shape=jax.ShapeDtypeStruct((n_pad, W1.shape[1]), jnp.float32),
    )(x_p, W1)

    dspec = _row_spec(DEG_W)
    y1 = pl.pallas_call(
        _prep_body, grid=grid,
        in_specs=[dspec, dspec, _row_spec(128)],
        out_specs=_row_spec(128),
        out_shape=jax.ShapeDtypeStruct((n_pad, 128), jnp.float32),
    )(d0, d1, xw1)

    agg = _sc_agg(n_pad, cpw)
    p0, p1 = agg(y1, src, dst, zeros_agg)

    xw2, y2 = pl.pallas_call(
        _fuse1_body, grid=grid,
        in_specs=[dspec, dspec, _row_spec(128), _row_spec(128), _row_spec(128),
                  _full_spec(b1r.shape), _full_spec(W2.shape)],
        out_specs=(_row_spec(128), _row_spec(128)),
        out_shape=(jax.ShapeDtypeStruct((n_pad, 128), jnp.float32),
                   jax.ShapeDtypeStruct((n_pad, 128), jnp.float32)),
    )(d0, d1, p0, p1, xw1, b1r, W2)

    q0, q1 = agg(y2, src, dst, zeros_agg)

    out = pl.pallas_call(
        _fuse2_body, grid=grid,
        in_specs=[dspec, dspec, _row_spec(128), _row_spec(128), _row_spec(128),
                  _full_spec(b2r.shape)],
        out_specs=_row_spec(128),
        out_shape=jax.ShapeDtypeStruct((n_pad, 128), jnp.float32),
    )(d0, d1, q0, q1, xw2, b2r)

    return out[:n]
